# baseline (speedup 1.0000x reference)
import functools

import jax
import jax.numpy as jnp
from jax import lax
from jax.experimental import pallas as pl
from jax.experimental.pallas import tpu as pltpu
from jax.experimental.pallas import tpu_sc as plsc

B = 16384
D = 32
SPAN = 128
W = 4
NBUF = 4
STG = 128

_NC, _NS = 2, 16
_NW = _NC * _NS
_BPW = B // _NW
_NWAVE = _BPW // W


def _make_gather_body(bpw):
    nwave = bpw // W

    def _gather_body(uidx_hbm, midx_hbm, utab_hbm, mtab_hbm, ue_hbm, me_hbm,
                     idx_v, span, gbuf, sem):
        wid = lax.axis_index("s") * _NC + lax.axis_index("c")
        base = wid * bpw
        lane = lax.iota(jnp.int32, 16)

        for idx_hbm, tab_hbm, out_hbm in (
            (uidx_hbm, utab_hbm, ue_hbm),
            (midx_hbm, mtab_hbm, me_hbm),
        ):
            pltpu.sync_copy(idx_hbm.at[wid], idx_v.at[pl.ds(0, bpw)])

            def fire(vec, b):
                for k in range(W):
                    al = pl.multiple_of(
                        lax.shift_right_logical(vec[k], 7) * SPAN, SPAN)
                    pltpu.async_copy(tab_hbm.at[:, pl.ds(al, SPAN)],
                                     span.at[b, k], sem)

            def body(w, _):
                b = lax.rem(w, NBUF)
                vec = idx_v[pl.ds(w * W, 16)]

                @pl.when(w + NBUF - 1 < nwave)
                def _():
                    fire(idx_v[pl.ds((w + NBUF - 1) * W, 16)],
                         lax.rem(w + NBUF - 1, NBUF))

                for k in range(W):
                    pltpu.make_async_copy(tab_hbm.at[:, pl.ds(0, SPAN)],
                                          span.at[b, k], sem).wait()
                for k in range(W):
                    r = lax.rem(w * W + k, STG)
                    lo = lax.broadcast_in_dim(
                        lax.bitwise_and(vec[k], SPAN - 1), (16,), ())
                    gbuf[r, pl.ds(0, 16)] = plsc.load_gather(
                        span.at[b, k], [lane, lo])
                    gbuf[r, pl.ds(16, 16)] = plsc.load_gather(
                        span.at[b, k], [lane + 16, lo])

                @pl.when(lax.rem(w, STG // W) == STG // W - 1)
                def _():
                    blk = lax.div(w, STG // W)
                    pltpu.sync_copy(
                        gbuf, out_hbm.at[pl.ds(base + blk * STG, STG)])
                return 0

            for p in range(NBUF - 1):
                fire(idx_v[pl.ds(p * W, 16)], p)
            lax.fori_loop(0, nwave, body, 0)

    return _gather_body


@functools.cache
def _gather(n):
    bpw = n // _NW
    return pl.kernel(
        _make_gather_body(bpw),
        mesh=plsc.VectorSubcoreMesh(core_axis_name="c", subcore_axis_name="s"),
        out_type=(
            jax.ShapeDtypeStruct((n, D), jnp.float32),
            jax.ShapeDtypeStruct((n, D), jnp.float32),
        ),
        scratch_types=[
            pltpu.VMEM((bpw + 32, ), jnp.int32),
            pltpu.VMEM((NBUF, W, D, SPAN), jnp.float32),
            pltpu.VMEM((STG, D), jnp.float32),
            pltpu.SemaphoreType.DMA,
        ],
        compiler_params=pltpu.CompilerParams(needs_layout_passes=False),
    )


def _mlp_body(ue, me, w1u, w1m, b1, w2, b2, w3, b3, out):
    h = jnp.dot(ue[...], w1u[...], preferred_element_type=jnp.float32)
    h = h + jnp.dot(me[...], w1m[...], preferred_element_type=jnp.float32)
    h = jnp.maximum(h + b1[...], 0.0)
    h = jnp.maximum(jnp.dot(h, w2[...], preferred_element_type=jnp.float32) + b2[...], 0.0)
    out[...] = jnp.dot(h, w3[...], preferred_element_type=jnp.float32) + b3[...]


def _mlp(ue, me, w1u, w1m, b1, w2, b2, w3, b3):
    n = ue.shape[0]
    blk = 2048
    rep = lambda i: (0, 0)
    return pl.pallas_call(
        _mlp_body,
        grid=(n // blk,),
        in_specs=[
            pl.BlockSpec((blk, D), lambda i: (i, 0)),
            pl.BlockSpec((blk, D), lambda i: (i, 0)),
            pl.BlockSpec((D, 256), rep),
            pl.BlockSpec((D, 256), rep),
            pl.BlockSpec((1, 256), rep),
            pl.BlockSpec((256, 64), rep),
            pl.BlockSpec((1, 64), rep),
            pl.BlockSpec((64, 1), rep),
            pl.BlockSpec((1, 1), rep),
        ],
        out_specs=pl.BlockSpec((blk, 1), lambda i: (i, 0)),
        out_shape=jax.ShapeDtypeStruct((n, 1), jnp.float32),
    )(ue, me, w1u, w1m, b1, w2, b2, w3, b3)


def kernel(user_id, movie_title, user_table, movie_table, W1, b1, W2, b2, W3, b3):
    half = B // 2
    uid = user_id.astype(jnp.int32)
    mid = movie_title.astype(jnp.int32)
    utT, mtT = user_table.T, movie_table.T
    outs = []
    for h in range(2):
        u = lax.dynamic_slice_in_dim(uid, h * half, half).reshape(_NW, -1)
        m = lax.dynamic_slice_in_dim(mid, h * half, half).reshape(_NW, -1)
        ue, me = _gather(half)(u, m, utT, mtT)
        outs.append(_mlp(ue, me, W1[:D], W1[D:], b1.reshape(1, -1),
                         W2, b2.reshape(1, -1), W3, b3.reshape(1, 1)))
    return jnp.concatenate(outs, axis=0)


# speedup vs baseline: 1.0110x; 1.0110x over previous
import functools

import jax
import jax.numpy as jnp
from jax import lax
from jax.experimental import pallas as pl
from jax.experimental.pallas import tpu as pltpu
from jax.experimental.pallas import tpu_sc as plsc

B = 16384
D = 32
SPAN = 128
W = 4
NBUF = 4
STG = 128

_NC, _NS = 2, 16
_NW = _NC * _NS
_BPW = B // _NW
_NWAVE = _BPW // W


def _gather_body(uidx_hbm, midx_hbm, utab_hbm, mtab_hbm, ue_hbm, me_hbm,
                 idx_v, span, gbuf, sem):
    wid = lax.axis_index("s") * _NC + lax.axis_index("c")
    base = wid * _BPW
    lane = lax.iota(jnp.int32, 16)

    for idx_hbm, tab_hbm, out_hbm in (
        (uidx_hbm, utab_hbm, ue_hbm),
        (midx_hbm, mtab_hbm, me_hbm),
    ):
        pltpu.sync_copy(idx_hbm.at[wid], idx_v.at[pl.ds(0, _BPW)])

        def fire(vec, b):
            for k in range(W):
                al = pl.multiple_of(
                    lax.shift_right_logical(vec[k], 7) * SPAN, SPAN)
                pltpu.async_copy(tab_hbm.at[:, pl.ds(al, SPAN)],
                                 span.at[b, k], sem)

        def body(w, _):
            b = lax.rem(w, NBUF)
            vec = idx_v[pl.ds(w * W, 16)]

            @pl.when(w + NBUF - 1 < _NWAVE)
            def _():
                fire(idx_v[pl.ds((w + NBUF - 1) * W, 16)],
                     lax.rem(w + NBUF - 1, NBUF))

            for k in range(W):
                pltpu.make_async_copy(tab_hbm.at[:, pl.ds(0, SPAN)],
                                      span.at[b, k], sem).wait()
            for k in range(W):
                r = lax.rem(w * W + k, STG)
                lo = lax.broadcast_in_dim(
                    lax.bitwise_and(vec[k], SPAN - 1), (16,), ())
                gbuf[r, pl.ds(0, 16)] = plsc.load_gather(
                    span.at[b, k], [lane, lo])
                gbuf[r, pl.ds(16, 16)] = plsc.load_gather(
                    span.at[b, k], [lane + 16, lo])

            @pl.when(lax.rem(w, STG // W) == STG // W - 1)
            def _():
                blk = lax.div(w, STG // W)
                pltpu.sync_copy(
                    gbuf, out_hbm.at[pl.ds(base + blk * STG, STG)])
            return 0

        for p in range(NBUF - 1):
            fire(idx_v[pl.ds(p * W, 16)], p)
        lax.fori_loop(0, _NWAVE, body, 0)


@functools.cache
def _gather():
    return pl.kernel(
        _gather_body,
        mesh=plsc.VectorSubcoreMesh(core_axis_name="c", subcore_axis_name="s"),
        out_type=(
            jax.ShapeDtypeStruct((B, D), jnp.float32),
            jax.ShapeDtypeStruct((B, D), jnp.float32),
        ),
        scratch_types=[
            pltpu.VMEM((_BPW + 32, ), jnp.int32),
            pltpu.VMEM((NBUF, W, D, SPAN), jnp.float32),
            pltpu.VMEM((STG, D), jnp.float32),
            pltpu.SemaphoreType.DMA,
        ],
        compiler_params=pltpu.CompilerParams(needs_layout_passes=False),
    )


def _mlp_body(ue, me, w1u, w1m, b1, w2, b2, w3, b3, out):
    h = jnp.dot(ue[...], w1u[...], preferred_element_type=jnp.float32)
    h = h + jnp.dot(me[...], w1m[...], preferred_element_type=jnp.float32)
    h = jnp.maximum(h + b1[...], 0.0)
    h = jnp.maximum(jnp.dot(h, w2[...], preferred_element_type=jnp.float32) + b2[...], 0.0)
    out[...] = jnp.dot(h, w3[...], preferred_element_type=jnp.float32) + b3[...]


def _mlp(ue, me, w1u, w1m, b1, w2, b2, w3, b3):
    blk = 2048
    rep = lambda i: (0, 0)
    return pl.pallas_call(
        _mlp_body,
        grid=(B // blk,),
        in_specs=[
            pl.BlockSpec((blk, D), lambda i: (i, 0)),
            pl.BlockSpec((blk, D), lambda i: (i, 0)),
            pl.BlockSpec((D, 256), rep),
            pl.BlockSpec((D, 256), rep),
            pl.BlockSpec((1, 256), rep),
            pl.BlockSpec((256, 64), rep),
            pl.BlockSpec((1, 64), rep),
            pl.BlockSpec((64, 1), rep),
            pl.BlockSpec((1, 1), rep),
        ],
        out_specs=pl.BlockSpec((blk, 1), lambda i: (i, 0)),
        out_shape=jax.ShapeDtypeStruct((B, 1), jnp.float32),
    )(ue, me, w1u, w1m, b1, w2, b2, w3, b3)


def kernel(user_id, movie_title, user_table, movie_table, W1, b1, W2, b2, W3, b3):
    uidx = user_id.astype(jnp.int32).reshape(_NW, _BPW)
    midx = movie_title.astype(jnp.int32).reshape(_NW, _BPW)
    ue, me = _gather()(uidx, midx, user_table.T, movie_table.T)
    return _mlp(ue, me, W1[:D], W1[D:], b1.reshape(1, -1),
                W2, b2.reshape(1, -1), W3, b3.reshape(1, 1))
